# baseline (speedup 1.0000x reference)
import functools

import numpy as np
import jax
import jax.numpy as jnp
from jax import lax
from jax.experimental import pallas as pl
from jax.experimental.pallas import tpu as pltpu
from jax.experimental.pallas import tpu_sc as plsc

_N_BINS = 15
_C = 32
_STEP = np.float32(1.0) / np.float32(15.0)

_NC = 2
_NS = 16
_NW = _NC * _NS
_CHUNK = 16384
_UNROLL = 8
_ACC = 1536


def _sc_body(n_per_tile, lg, tg, out, lb0, lb1, tb0, tb1,
             acc0, acc1, acc2, acc3, ls0, ls1, ts0, ts1):
    wid = lax.axis_index("s") * _NC + lax.axis_index("c")
    base = wid * n_per_tile
    iota = lax.iota(jnp.int32, 16)
    ones = jnp.full((16,), 1.0, jnp.float32)
    zeros = jnp.zeros((16,), jnp.float32)

    accs = (acc0, acc1, acc2, acc3)
    for a in accs:
        for i in range(_ACC // 16):
            a[pl.ds(i * 16, 16)] = zeros

    lbufs = (lb0, lb1)
    tbufs = (tb0, tb1)
    lsems = (ls0, ls1)
    tsems = (ts0, ts1)

    def start(k, b):
        off = base + k * _CHUNK
        pltpu.async_copy(lg.at[pl.ds(off, _CHUNK)], lbufs[b], lsems[b])
        pltpu.async_copy(tg.at[pl.ds(off, _CHUNK)], tbufs[b], tsems[b])

    def wait(k, b):
        off = base + k * _CHUNK
        pltpu.make_async_copy(lg.at[pl.ds(off, _CHUNK)], lbufs[b],
                              lsems[b]).wait()
        pltpu.make_async_copy(tg.at[pl.ds(off, _CHUNK)], tbufs[b],
                              tsems[b]).wait()

    def process(b):
        lb = lbufs[b]
        tb = tbufs[b]

        @plsc.parallel_loop(0, _CHUNK // 16, 2, unroll=_UNROLL)
        def _(v):
            o = v * 16
            for j in range(2):
                cvec = iota if j == 0 else iota + 16
                x = lb[pl.ds(o + j * 16, 16)]
                tv = tb[pl.ds(o + j * 16, 16)]
                p = 1.0 / (1.0 + jnp.exp(-x))
                b0 = jnp.minimum((p * 15.0).astype(jnp.int32), 14)
                b0f = b0.astype(jnp.float32)
                blo = b0f * _STEP
                bhi = (b0f + 1.0) * _STEP
                adj = jnp.where(p >= bhi, 1, 0) - jnp.where(p < blo, 1, 0)
                bi = jnp.minimum(b0 + adj, 14)
                idx = (bi << 5) + cvec
                plsc.addupdate_scatter(accs[2 * j], [idx + (tv << 9)], ones)
                plsc.addupdate_scatter(accs[2 * j + 1], [idx + 1024], p)

    nch = n_per_tile // _CHUNK
    start(0, 0)
    start(1, 1)

    def outer(i, carry):
        k0 = i * 2
        wait(k0, 0)
        process(0)

        @pl.when(k0 + 2 < nch)
        def _():
            start(k0 + 2, 0)

        wait(k0 + 1, 1)
        process(1)

        @pl.when(k0 + 3 < nch)
        def _():
            start(k0 + 3, 1)

        return carry

    lax.fori_loop(0, nch // 2, outer, 0)
    for i in range(_ACC // 16):
        sl = pl.ds(i * 16, 16)
        acc0[sl] = (acc0[sl] + acc1[sl]) + (acc2[sl] + acc3[sl])
    pltpu.sync_copy(acc0, out.at[wid])


@functools.cache
def _make_sc_partials(n_total):
    n_per_tile = n_total // _NW
    assert n_per_tile % (2 * _CHUNK) == 0

    return pl.kernel(
        functools.partial(_sc_body, n_per_tile),
        out_type=jax.ShapeDtypeStruct((_NW, _ACC), jnp.float32),
        mesh=plsc.VectorSubcoreMesh(core_axis_name="c", subcore_axis_name="s",
                                    num_cores=_NC, num_subcores=_NS),
        compiler_params=pltpu.CompilerParams(needs_layout_passes=False),
        scratch_types=[
            pltpu.VMEM((_CHUNK,), jnp.float32),
            pltpu.VMEM((_CHUNK,), jnp.float32),
            pltpu.VMEM((_CHUNK,), jnp.int32),
            pltpu.VMEM((_CHUNK,), jnp.int32),
            pltpu.VMEM((_ACC,), jnp.float32),
            pltpu.VMEM((_ACC,), jnp.float32),
            pltpu.VMEM((_ACC,), jnp.float32),
            pltpu.VMEM((_ACC,), jnp.float32),
            pltpu.SemaphoreType.DMA,
            pltpu.SemaphoreType.DMA,
            pltpu.SemaphoreType.DMA,
            pltpu.SemaphoreType.DMA,
        ],
    )


def _finish_body(n_rows, x_ref, o_ref):
    x = x_ref[:]
    s = jnp.sum(x, axis=0, keepdims=True)
    a0 = s[:, 0:512]
    a1 = s[:, 512:1024]
    bs = s[:, 1024:1536]
    cnt = a0 + a1
    nonempty = cnt > 0
    safe = jnp.where(nonempty, cnt, 1.0)
    w = cnt / np.float32(n_rows)
    terms = jnp.where(nonempty, jnp.abs(bs / safe - a1 / safe) * w, 0.0)
    ece = jnp.sum(terms)
    count = jnp.sum(jnp.where(nonempty, 1.0, 0.0))
    o_ref[...] = jnp.broadcast_to(jnp.where(count > 0, ece / count, ece),
                                  (1, 1))


@jax.jit
def _impl(logits, targets):
    n_rows, n_cols = logits.shape
    assert n_cols == _C
    partials = _make_sc_partials(n_rows * n_cols)(
        logits.reshape(-1), targets.reshape(-1))
    out = pl.pallas_call(
        functools.partial(_finish_body, n_rows),
        out_shape=jax.ShapeDtypeStruct((1, 1), jnp.float32),
    )(partials)
    return out[0, 0]


def kernel(logits, targets):
    return _impl(logits, targets)


# speedup vs baseline: 1.0143x; 1.0143x over previous
import functools

import numpy as np
import jax
import jax.numpy as jnp
from jax import lax
from jax.experimental import pallas as pl
from jax.experimental.pallas import tpu as pltpu
from jax.experimental.pallas import tpu_sc as plsc

_N_BINS = 15
_C = 32
_STEP = np.float32(1.0) / np.float32(15.0)

_NC = 2
_NS = 16
_NW = _NC * _NS
_CHUNK = 16384
_CROWS = _CHUNK // _C
_UNROLL = 4
_ACC = 1536


def _sc_body(rows_per_tile, lg, tg, out, lb0, lb1, tb0, tb1,
             acc0, acc1, acc2, acc3, ls0, ls1, ts0, ts1):
    wid = lax.axis_index("s") * _NC + lax.axis_index("c")
    base = wid * rows_per_tile
    iota = lax.iota(jnp.int32, 16)
    ones = jnp.full((16,), 1.0, jnp.float32)
    zeros = jnp.zeros((16,), jnp.float32)

    accs = (acc0, acc1, acc2, acc3)
    for a in accs:
        for i in range(_ACC // 16):
            a[pl.ds(i * 16, 16)] = zeros

    lbufs = (lb0, lb1)
    tbufs = (tb0, tb1)
    lsems = (ls0, ls1)
    tsems = (ts0, ts1)

    def start(k, b):
        off = base + k * _CROWS
        pltpu.async_copy(lg.at[pl.ds(off, _CROWS), :], lbufs[b], lsems[b])
        pltpu.async_copy(tg.at[pl.ds(off, _CROWS), :], tbufs[b], tsems[b])

    def wait(k, b):
        off = base + k * _CROWS
        pltpu.make_async_copy(lg.at[pl.ds(off, _CROWS), :], lbufs[b],
                              lsems[b]).wait()
        pltpu.make_async_copy(tg.at[pl.ds(off, _CROWS), :], tbufs[b],
                              tsems[b]).wait()

    def process(b):
        lb = lbufs[b]
        tb = tbufs[b]

        @plsc.parallel_loop(0, _CHUNK // 16, 2, unroll=_UNROLL)
        def _(v):
            r = v >> 1
            for j in range(2):
                cvec = iota if j == 0 else iota + 16
                x = lb[r, pl.ds(j * 16, 16)]
                tv = tb[r, pl.ds(j * 16, 16)]
                p = 1.0 / (1.0 + jnp.exp(-x))
                b0 = jnp.minimum((p * 15.0).astype(jnp.int32), 14)
                b0f = b0.astype(jnp.float32)
                blo = b0f * _STEP
                bhi = (b0f + 1.0) * _STEP
                adj = jnp.where(p >= bhi, 1, 0) - jnp.where(p < blo, 1, 0)
                bi = jnp.minimum(b0 + adj, 14)
                idx = (bi << 5) + cvec
                plsc.addupdate_scatter(accs[2 * j], [idx + (tv << 9)], ones)
                plsc.addupdate_scatter(accs[2 * j + 1], [idx + 1024], p)

    nch = rows_per_tile // _CROWS
    start(0, 0)
    start(1, 1)

    def outer(i, carry):
        k0 = i * 2
        wait(k0, 0)
        process(0)

        @pl.when(k0 + 2 < nch)
        def _():
            start(k0 + 2, 0)

        wait(k0 + 1, 1)
        process(1)

        @pl.when(k0 + 3 < nch)
        def _():
            start(k0 + 3, 1)

        return carry

    lax.fori_loop(0, nch // 2, outer, 0)
    for i in range(_ACC // 16):
        sl = pl.ds(i * 16, 16)
        acc0[sl] = (acc0[sl] + acc1[sl]) + (acc2[sl] + acc3[sl])
    pltpu.sync_copy(acc0, out.at[wid])


@functools.cache
def _make_sc_partials(n_rows):
    rows_per_tile = n_rows // _NW
    assert rows_per_tile % (2 * _CROWS) == 0

    return pl.kernel(
        functools.partial(_sc_body, rows_per_tile),
        out_type=jax.ShapeDtypeStruct((_NW, _ACC), jnp.float32),
        mesh=plsc.VectorSubcoreMesh(core_axis_name="c", subcore_axis_name="s",
                                    num_cores=_NC, num_subcores=_NS),
        compiler_params=pltpu.CompilerParams(needs_layout_passes=False,
                                             use_tc_tiling_on_sc=False),
        scratch_types=[
            pltpu.VMEM((_CROWS, _C), jnp.float32),
            pltpu.VMEM((_CROWS, _C), jnp.float32),
            pltpu.VMEM((_CROWS, _C), jnp.int32),
            pltpu.VMEM((_CROWS, _C), jnp.int32),
            pltpu.VMEM((_ACC,), jnp.float32),
            pltpu.VMEM((_ACC,), jnp.float32),
            pltpu.VMEM((_ACC,), jnp.float32),
            pltpu.VMEM((_ACC,), jnp.float32),
            pltpu.SemaphoreType.DMA,
            pltpu.SemaphoreType.DMA,
            pltpu.SemaphoreType.DMA,
            pltpu.SemaphoreType.DMA,
        ],
    )


def _finish_body(n_rows, x_ref, o_ref):
    x = x_ref[:]
    s = jnp.sum(x, axis=0, keepdims=True)
    a0 = s[:, 0:512]
    a1 = s[:, 512:1024]
    bs = s[:, 1024:1536]
    cnt = a0 + a1
    nonempty = cnt > 0
    safe = jnp.where(nonempty, cnt, 1.0)
    w = cnt / np.float32(n_rows)
    terms = jnp.where(nonempty, jnp.abs(bs / safe - a1 / safe) * w, 0.0)
    ece = jnp.sum(terms)
    count = jnp.sum(jnp.where(nonempty, 1.0, 0.0))
    o_ref[...] = jnp.broadcast_to(jnp.where(count > 0, ece / count, ece),
                                  (1, 1))


@jax.jit
def _impl(logits, targets):
    n_rows, n_cols = logits.shape
    assert n_cols == _C
    partials = _make_sc_partials(n_rows)(logits, targets)
    out = pl.pallas_call(
        functools.partial(_finish_body, n_rows),
        out_shape=jax.ShapeDtypeStruct((1, 1), jnp.float32),
    )(partials)
    return out[0, 0]


def kernel(logits, targets):
    return _impl(logits, targets)


# speedup vs baseline: 1.2214x; 1.2042x over previous
import functools

import numpy as np
import jax
import jax.numpy as jnp
from jax import lax
from jax.experimental import pallas as pl
from jax.experimental.pallas import tpu as pltpu
from jax.experimental.pallas import tpu_sc as plsc

_N_BINS = 15
_C = 32
_STEP = np.float32(1.0) / np.float32(15.0)

_NC = 2
_NS = 16
_NW = _NC * _NS
_CHUNK = 16384
_CROWS = 128
_UNROLL = 4
_ACC = 1536


def _sc_body(rows_per_tile, lg, tg, out, lb0, lb1, tb0, tb1,
             acc0, acc1, acc2, acc3, ls0, ls1, ts0, ts1):
    wid = lax.axis_index("s") * _NC + lax.axis_index("c")
    base = wid * rows_per_tile
    iota = lax.iota(jnp.int32, 16)
    ones = jnp.full((16,), 1.0, jnp.float32)
    zeros = jnp.zeros((16,), jnp.float32)

    accs = (acc0, acc1, acc2, acc3)
    for a in accs:
        for i in range(_ACC // 16):
            a[pl.ds(i * 16, 16)] = zeros

    lbufs = (lb0, lb1)
    tbufs = (tb0, tb1)
    lsems = (ls0, ls1)
    tsems = (ts0, ts1)

    def start(k, b):
        off = base + k * _CROWS
        pltpu.async_copy(lg.at[pl.ds(off, _CROWS), :], lbufs[b], lsems[b])
        pltpu.async_copy(tg.at[pl.ds(off, _CROWS), :], tbufs[b], tsems[b])

    def wait(k, b):
        off = base + k * _CROWS
        pltpu.make_async_copy(lg.at[pl.ds(off, _CROWS), :], lbufs[b],
                              lsems[b]).wait()
        pltpu.make_async_copy(tg.at[pl.ds(off, _CROWS), :], tbufs[b],
                              tsems[b]).wait()

    def process(b):
        lb = lbufs[b]
        tb = tbufs[b]

        @plsc.parallel_loop(0, _CROWS * 2, 2, unroll=_UNROLL)
        def _(v):
            r = v >> 1
            for j in range(2):
                cvec = iota if j == 0 else iota + 16
                x = lb[r, pl.ds(j * 16, 16)]
                tv = tb[r, pl.ds(j * 16, 16)]
                p = 1.0 / (1.0 + jnp.exp(-x))
                b0 = jnp.minimum((p * 15.0).astype(jnp.int32), 14)
                b0f = b0.astype(jnp.float32)
                blo = b0f * _STEP
                bhi = (b0f + 1.0) * _STEP
                adj = jnp.where(p >= bhi, 1, 0) - jnp.where(p < blo, 1, 0)
                bi = jnp.minimum(b0 + adj, 14)
                idx = (bi << 5) + cvec
                plsc.addupdate_scatter(accs[2 * j], [idx + (tv << 9)], ones)
                plsc.addupdate_scatter(accs[2 * j + 1], [idx + 1024], p)

    nch = rows_per_tile // _CROWS
    start(0, 0)
    start(1, 1)

    def outer(i, carry):
        k0 = i * 2
        wait(k0, 0)
        process(0)

        @pl.when(k0 + 2 < nch)
        def _():
            start(k0 + 2, 0)

        wait(k0 + 1, 1)
        process(1)

        @pl.when(k0 + 3 < nch)
        def _():
            start(k0 + 3, 1)

        return carry

    lax.fori_loop(0, nch // 2, outer, 0)
    for i in range(_ACC // 16):
        sl = pl.ds(i * 16, 16)
        acc0[sl] = (acc0[sl] + acc1[sl]) + (acc2[sl] + acc3[sl])
    pltpu.sync_copy(acc0, out.at[wid])


@functools.cache
def _make_sc_partials(n_rows):
    rows_per_tile = n_rows // _NW
    assert rows_per_tile % (2 * _CROWS) == 0

    return pl.kernel(
        functools.partial(_sc_body, rows_per_tile),
        out_type=jax.ShapeDtypeStruct((_NW, _ACC), jnp.float32),
        mesh=plsc.VectorSubcoreMesh(core_axis_name="c", subcore_axis_name="s",
                                    num_cores=_NC, num_subcores=_NS),
        compiler_params=pltpu.CompilerParams(needs_layout_passes=False,
                                             use_tc_tiling_on_sc=True),
        scratch_types=[
            pltpu.VMEM((_CROWS, _C), jnp.float32),
            pltpu.VMEM((_CROWS, _C), jnp.float32),
            pltpu.VMEM((_CROWS, _C), jnp.int32),
            pltpu.VMEM((_CROWS, _C), jnp.int32),
            pltpu.VMEM((_ACC,), jnp.float32),
            pltpu.VMEM((_ACC,), jnp.float32),
            pltpu.VMEM((_ACC,), jnp.float32),
            pltpu.VMEM((_ACC,), jnp.float32),
            pltpu.SemaphoreType.DMA,
            pltpu.SemaphoreType.DMA,
            pltpu.SemaphoreType.DMA,
            pltpu.SemaphoreType.DMA,
        ],
    )


def _finish_body(n_rows, x_ref, o_ref):
    x = x_ref[:]
    s = jnp.sum(x, axis=0, keepdims=True)
    a0 = s[:, 0:512]
    a1 = s[:, 512:1024]
    bs = s[:, 1024:1536]
    cnt = a0 + a1
    nonempty = cnt > 0
    safe = jnp.where(nonempty, cnt, 1.0)
    w = cnt / np.float32(n_rows)
    terms = jnp.where(nonempty, jnp.abs(bs / safe - a1 / safe) * w, 0.0)
    ece = jnp.sum(terms)
    count = jnp.sum(jnp.where(nonempty, 1.0, 0.0))
    o_ref[...] = jnp.broadcast_to(jnp.where(count > 0, ece / count, ece),
                                  (1, 1))


@jax.jit
def _impl(logits, targets):
    n_rows, n_cols = logits.shape
    assert n_cols == _C
    partials = _make_sc_partials(n_rows)(logits, targets)
    out = pl.pallas_call(
        functools.partial(_finish_body, n_rows),
        out_shape=jax.ShapeDtypeStruct((1, 1), jnp.float32),
    )(partials)
    return out[0, 0]


def kernel(logits, targets):
    return _impl(logits, targets)
